# Initial kernel scaffold; baseline (speedup 1.0000x reference)
import jax
import jax.numpy as jnp
from jax.experimental import pallas as pl


def kernel(x, edge_index, batch, W0, W1, W2, l0w, l0b, l1w, l1b, l2w, l2b, rw):
    raise NotImplementedError("write your pallas kernel here")


# speedup vs baseline: 7.0929x; 7.0929x over previous
import jax
import jax.numpy as jnp
from jax import lax
from jax.experimental import pallas as pl
from jax.experimental.pallas import tpu as pltpu
from jax.experimental.pallas import tpu_sc as plsc

_N = 10000
_E = 320000
_D = 128
_G = 128
_T = 16

_NC = 2
_NS = 16
_NW = _NC * _NS
_CHUNK = 80
_EPW = _E // _NW
_NCHUNK = _EPW // _CHUNK
_RPT = 640
_RPT_LAST = _N - 15 * _RPT

_R = 2000
_NBLK = _N // _R


def _sc_aggregate(m, src2d, dst2d, zeros):
  mesh = plsc.VectorSubcoreMesh(core_axis_name="c", subcore_axis_name="s")

  def body(m_hbm, src_hbm, dst_hbm, z_hbm, out_hbm,
           src_v, dst_v, rows_v, acc, sem):
    cid = lax.axis_index("c")
    sid = lax.axis_index("s")
    wid = cid * _NS + sid
    @pl.when(sid < _NS - 1)
    def _zero_main():
      pltpu.sync_copy(z_hbm.at[pl.ds(sid * _RPT, _RPT)],
                      acc.at[pl.ds(sid * _RPT, _RPT)])

    @pl.when(sid == _NS - 1)
    def _zero_last():
      pltpu.sync_copy(z_hbm.at[pl.ds(15 * _RPT, _RPT_LAST)],
                      acc.at[pl.ds(15 * _RPT, _RPT_LAST)])

    pltpu.sync_copy(src_hbm.at[wid], src_v)
    pltpu.sync_copy(dst_hbm.at[wid], dst_v)
    plsc.subcore_barrier()

    def chunk(j, _):
      pltpu.async_copy(m_hbm.at[src_v.at[j]], rows_v, sem).wait()
      pltpu.sync_copy(rows_v, acc.at[dst_v.at[j]], add=True)
      return _

    lax.fori_loop(0, _NCHUNK, chunk, 0)
    plsc.subcore_barrier()

    @pl.when(sid < _NS - 1)
    def _out_main():
      pltpu.sync_copy(acc.at[pl.ds(sid * _RPT, _RPT)],
                      out_hbm.at[pl.ds(cid * _N + sid * _RPT, _RPT)])

    @pl.when(sid == _NS - 1)
    def _out_last():
      pltpu.sync_copy(acc.at[pl.ds(15 * _RPT, _RPT_LAST)],
                      out_hbm.at[pl.ds(cid * _N + 15 * _RPT, _RPT_LAST)])

  k = pl.kernel(
      body,
      mesh=mesh,
      out_type=jax.ShapeDtypeStruct((_NC * _N, _D), jnp.float32),
      scratch_types=[
          pltpu.VMEM((_NCHUNK, _CHUNK), jnp.int32),
          pltpu.VMEM((_NCHUNK, _CHUNK), jnp.int32),
          pltpu.VMEM((_CHUNK, _D), jnp.float32),
          pltpu.VMEM_SHARED((_N, _D), jnp.float32),
          pltpu.SemaphoreType.DMA,
      ],
  )
  return k(m, src2d, dst2d, zeros)


def _premsg_body(x_ref, lw_ref, lb_ref, o_ref):
  e = jax.nn.relu(lw_ref[...] + lb_ref[...])
  o_ref[...] = jax.nn.relu(x_ref[...] + e)


def _premsg(x, lw, lb):
  return pl.pallas_call(
      _premsg_body,
      out_shape=jax.ShapeDtypeStruct((_N, _D), jnp.float32),
  )(x, lw, lb)


def _layer_body(agg_ref, h_ref, w_ref, lw_ref, lb_ref, hn_ref, mn_ref):
  a = agg_ref[0] + agg_ref[1] + h_ref[...]
  hn = jax.nn.relu(jnp.dot(a, w_ref[...], preferred_element_type=jnp.float32))
  hn_ref[...] = hn
  e = jax.nn.relu(lw_ref[...] + lb_ref[...])
  mn_ref[...] = jax.nn.relu(hn + e)


def _layer(agg, h, W, lw_next, lb_next):
  return pl.pallas_call(
      _layer_body,
      grid=(_NBLK,),
      in_specs=[
          pl.BlockSpec((2, _R, _D), lambda i: (0, i, 0)),
          pl.BlockSpec((_R, _D), lambda i: (i, 0)),
          pl.BlockSpec((_D, _D), lambda i: (0, 0)),
          pl.BlockSpec((1, _D), lambda i: (0, 0)),
          pl.BlockSpec((1, _D), lambda i: (0, 0)),
      ],
      out_specs=[
          pl.BlockSpec((_R, _D), lambda i: (i, 0)),
          pl.BlockSpec((_R, _D), lambda i: (i, 0)),
      ],
      out_shape=[
          jax.ShapeDtypeStruct((_N, _D), jnp.float32),
          jax.ShapeDtypeStruct((_N, _D), jnp.float32),
      ],
  )(agg, h, W, lw_next, lb_next)


def _final_body(agg_ref, h_ref, w_ref, b_ref, rw_ref, o_ref, sums, cnts):
  i = pl.program_id(0)

  @pl.when(i == 0)
  def _init():
    sums[...] = jnp.zeros_like(sums)
    cnts[...] = jnp.zeros_like(cnts)

  a = agg_ref[0] + agg_ref[1] + h_ref[...]
  h3 = jax.nn.relu(jnp.dot(a, w_ref[...], preferred_element_type=jnp.float32))
  p = (b_ref[...] == lax.broadcasted_iota(jnp.int32, (1, _G), 1)
       ).astype(jnp.float32)
  sums[...] += lax.dot_general(p, h3, (((0,), (0,)), ((), ())),
                               preferred_element_type=jnp.float32)
  cnts[...] += lax.dot_general(p, jnp.ones((_R, _D), jnp.float32),
                               (((0,), (0,)), ((), ())),
                               preferred_element_type=jnp.float32)

  @pl.when(i == _NBLK - 1)
  def _readout():
    pooled = sums[...] / jnp.maximum(cnts[...], 1.0)
    o_ref[...] = jnp.dot(pooled, rw_ref[...],
                         preferred_element_type=jnp.float32)


def _final(agg, h, W, batch2d, rw):
  return pl.pallas_call(
      _final_body,
      grid=(_NBLK,),
      in_specs=[
          pl.BlockSpec((2, _R, _D), lambda i: (0, i, 0)),
          pl.BlockSpec((_R, _D), lambda i: (i, 0)),
          pl.BlockSpec((_D, _D), lambda i: (0, 0)),
          pl.BlockSpec((_R, 1), lambda i: (i, 0)),
          pl.BlockSpec((_D, _T), lambda i: (0, 0)),
      ],
      out_specs=pl.BlockSpec((_G, _T), lambda i: (0, 0)),
      out_shape=jax.ShapeDtypeStruct((_G, _T), jnp.float32),
      scratch_shapes=[
          pltpu.VMEM((_G, _D), jnp.float32),
          pltpu.VMEM((_G, _D), jnp.float32),
      ],
  )(agg, h, W, batch2d, rw)


def kernel(x, edge_index, batch, W0, W1, W2,
           l0w, l0b, l1w, l1b, l2w, l2b, rw):
  src2d = edge_index[0].reshape(_NW, _NCHUNK, _CHUNK)
  dst2d = edge_index[1].reshape(_NW, _NCHUNK, _CHUNK)
  zeros = jnp.zeros((_N, _D), jnp.float32)
  batch2d = batch.reshape(_N, 1)
  l0b2, l1b2, l2b2 = (l0b.reshape(1, _D), l1b.reshape(1, _D),
                      l2b.reshape(1, _D))

  m0 = _premsg(x, l0w, l0b2)
  agg0 = _sc_aggregate(m0, src2d, dst2d, zeros).reshape(2, _N, _D)
  h1, m1 = _layer(agg0, x, W0, l1w, l1b2)
  agg1 = _sc_aggregate(m1, src2d, dst2d, zeros).reshape(2, _N, _D)
  h2, m2 = _layer(agg1, h1, W1, l2w, l2b2)
  agg2 = _sc_aggregate(m2, src2d, dst2d, zeros).reshape(2, _N, _D)
  return _final(agg2, h2, W2, batch2d, rw)
